# baseline (speedup 1.0000x reference)
import functools
import math

import jax
import jax.numpy as jnp
from jax.experimental import pallas as pl
from jax.experimental.pallas import tpu as pltpu

B, C, T = 32, 64, 96
PATCH = 12
HEADS = 4
FF = 64
HD = FF // HEADS
TOPK = 8
ALPHA = 0.5
EPS = 1e-5
NBLK = T // PATCH
PQ = PATCH // 2


def _gelu(x):
    return 0.5 * x * (1.0 + jax.lax.erf(x * (1.0 / math.sqrt(2.0))))


def _body(xt_ref, pk1_ref, pk2_ref, out_ref):
    xt = xt_ref[:]

    ng_r = pk1_ref[0:T, :]
    nb_r = pk1_ref[T:2 * T, :]
    n1g = pk1_ref[192:204, :][None]
    n1b = pk1_ref[204:216, :][None]
    n2g = pk1_ref[216:228, :][None]
    n2b = pk1_ref[228:240, :][None]
    wa_e = pk1_ref[240:252, 0:PATCH][None, :, :, None]
    ba_e = pk1_ref[252:253, 0:PATCH].T[None]
    bm2_e = pk2_ref[8:9, 7:8]

    mu = jnp.mean(xt, axis=0, keepdims=True)
    d = xt - mu
    var = jnp.mean(d * d, axis=0, keepdims=True)
    xn = d / jnp.sqrt(var + EPS) * ng_r[None] + nb_r[None]

    blocks = [xn[:, 0:PATCH, :]]
    prev = blocks[0]
    for k in range(1, NBLK):
        xk = xn[:, k * PATCH:(k + 1) * PATCH, :]

        m1 = jnp.mean(prev, axis=0, keepdims=True)
        d1 = prev - m1
        v1 = jnp.mean(d1 * d1, axis=0, keepdims=True)
        inp = d1 / jnp.sqrt(v1 + EPS) * n1g + n1b

        agg = jnp.sum(inp[:, None, :, :] * wa_e, axis=2)
        agg = _gelu(agg + ba_e)
        tmp = agg + xk
        res = tmp

        m2 = jnp.mean(tmp, axis=0, keepdims=True)
        d2 = tmp - m2
        v2 = jnp.mean(d2 * d2, axis=0, keepdims=True)
        s = d2 / jnp.sqrt(v2 + EPS) * n2g + n2b

        st = s.reshape(B * PATCH, C).T
        iota = jax.lax.broadcasted_iota(jnp.int32, (C, B * PATCH), 0)
        cur = st
        tops = []
        for _ in range(TOPK):
            mj = jnp.max(cur, axis=0, keepdims=True)
            tops.append(mj)
            am = jnp.min(jnp.where(cur == mj, iota, C), axis=0, keepdims=True)
            cur = jnp.where(iota == am, -1e30, cur)
        cur = st
        bots = []
        for _ in range(TOPK):
            mj = jnp.min(cur, axis=0, keepdims=True)
            bots.append(mj)
            am = jnp.min(jnp.where(cur == mj, iota, C), axis=0, keepdims=True)
            cur = jnp.where(iota == am, 1e30, cur)
        t8s = jnp.concatenate(tops, axis=0)[:, None, :]
        b8s = jnp.concatenate(bots, axis=0)[:, None, :]

        pre = pk2_ref[:, 4:5][:, :, None]
        for h in range(HEADS):
            a_h = pk2_ref[h:h + 1, 6:7]
            c_h = pk2_ref[h + 4:h + 5, 6:7]
            g = a_h * st + c_h
            vsel = jnp.where(g[None] > 0.0, t8s, b8s)
            z = g[None] * vsel
            e = jnp.exp(z)
            den = jnp.sum(e, axis=0)
            num = jnp.sum(e * vsel, axis=0)
            m_h = num / den
            pre = pre + m_h[None] * pk2_ref[:, h:h + 1][:, :, None]

        gact = _gelu(pre)
        z_t = jnp.sum(gact * pk2_ref[:, 5:6][:, :, None], axis=0) + bm2_e
        prev = res + ALPHA * z_t.T.reshape(B, PATCH, C)
        blocks.append(prev)

    out_ref[:] = jnp.concatenate(blocks, axis=1)


@jax.jit
def kernel(x, ng, nb, n1g, n1b, n2g, n2b, Wa, ba, Wce, bce, Wq, bq,
           Wk, bk, Wv, bv, Wm1, bm1, Wm2, bm2):
    f32 = jnp.float32
    xt = jnp.transpose(x, (0, 2, 1))
    ng2 = ng.reshape(C, T).T
    nb2 = nb.reshape(C, T).T
    n1g2 = n1g.reshape(C, PATCH).T
    n1b2 = n1b.reshape(C, PATCH).T
    n2g2 = n2g.reshape(C, PATCH).T
    n2b2 = n2b.reshape(C, PATCH).T

    w = Wce[:, 0]
    uq = Wq @ w
    cq = Wq @ bce + bq
    uk = Wk @ w
    ck = Wk @ bce + bk
    uv = Wv @ w
    cv = Wv @ bce + bv
    scale = 1.0 / math.sqrt(HD)
    uqh = uq.reshape(HEADS, HD)
    ukh = uk.reshape(HEADS, HD)
    cqh = cq.reshape(HEADS, HD)
    uvh = uv.reshape(HEADS, HD)
    a_h = jnp.sum(uqh * ukh, axis=1) * scale
    c_h = jnp.sum(cqh * ukh, axis=1) * scale
    at = jnp.sum(Wm1.reshape(FF, HEADS, HD) * uvh[None], axis=2)
    c0 = Wm1 @ cv + bm1
    wm2 = Wm2[0]

    wa_pad = jnp.zeros((PATCH, C), f32).at[:, 0:PATCH].set(Wa)
    ba_pad = jnp.zeros((1, C), f32).at[0, 0:PATCH].set(ba)
    pk1 = jnp.concatenate(
        [ng2, nb2, n1g2, n1b2, n2g2, n2b2, wa_pad, ba_pad,
         jnp.zeros((2, C), f32)], axis=0)
    ahch = jnp.zeros((FF,), f32).at[0:HEADS].set(a_h).at[HEADS:2 * HEADS].set(c_h)
    bm2c = jnp.zeros((FF,), f32).at[8].set(bm2[0])
    pk2 = jnp.stack([at[:, 0], at[:, 1], at[:, 2], at[:, 3],
                     c0, wm2, ahch, bm2c], axis=1)
    out = pl.pallas_call(
        _body,
        out_shape=jax.ShapeDtypeStruct((B, T, C), f32),
    )(xt.astype(f32), pk1, pk2)
    return jnp.transpose(out, (0, 2, 1))


# speedup vs baseline: 1.0266x; 1.0266x over previous
import functools
import math

import jax
import jax.numpy as jnp
from jax.experimental import pallas as pl
from jax.experimental.pallas import tpu as pltpu

B, C, T = 32, 64, 96
PATCH = 12
HEADS = 4
FF = 64
HD = FF // HEADS
TOPK = 8
ALPHA = 0.5
EPS = 1e-5
NBLK = T // PATCH


def _gelu(x):
    return 0.5 * x * (1.0 + jax.lax.erf(x * (1.0 / math.sqrt(2.0))))


def _body(xt_ref, ng_ref, nb_ref, n1g_ref, n1b_ref, n2g_ref, n2b_ref,
          wa_ref, ba_ref, ah_ref, ch_ref, at_ref, c0_ref, wm2_ref, bm2_ref,
          out_ref):
    xt = xt_ref[:]

    mu = jnp.mean(xt, axis=0, keepdims=True)
    d = xt - mu
    var = jnp.mean(d * d, axis=0, keepdims=True)
    xn = d / jnp.sqrt(var + EPS) * ng_ref[:][None] + nb_ref[:][None]

    n1g = n1g_ref[:][None]
    n1b = n1b_ref[:][None]
    n2g = n2g_ref[:][None]
    n2b = n2b_ref[:][None]
    wa_e = wa_ref[:][None, :, :, None]
    ba_e = ba_ref[:][None]
    bm2_e = bm2_ref[:]

    blocks = [xn[:, 0:PATCH, :]]
    prev = blocks[0]
    for k in range(1, NBLK):
        xk = xn[:, k * PATCH:(k + 1) * PATCH, :]

        m1 = jnp.mean(prev, axis=0, keepdims=True)
        d1 = prev - m1
        v1 = jnp.mean(d1 * d1, axis=0, keepdims=True)
        inp = d1 / jnp.sqrt(v1 + EPS) * n1g + n1b

        agg = jnp.sum(inp[:, None, :, :] * wa_e, axis=2)
        agg = _gelu(agg + ba_e)
        tmp = agg + xk
        res = tmp

        m2 = jnp.mean(tmp, axis=0, keepdims=True)
        d2 = tmp - m2
        v2 = jnp.mean(d2 * d2, axis=0, keepdims=True)
        s = d2 / jnp.sqrt(v2 + EPS) * n2g + n2b

        st = s.reshape(B * PATCH, C).T
        iota = jax.lax.broadcasted_iota(jnp.int32, (C, B * PATCH), 0)
        cur = st
        tops = []
        for _ in range(TOPK):
            mj = jnp.max(cur, axis=0, keepdims=True)
            tops.append(mj)
            am = jnp.min(jnp.where(cur == mj, iota, C), axis=0, keepdims=True)
            cur = jnp.where(iota == am, -1e30, cur)
        cur = st
        bots = []
        for _ in range(TOPK):
            mj = jnp.min(cur, axis=0, keepdims=True)
            bots.append(mj)
            am = jnp.min(jnp.where(cur == mj, iota, C), axis=0, keepdims=True)
            cur = jnp.where(iota == am, 1e30, cur)
        t8s = jnp.concatenate(tops, axis=0)[:, None, :]
        b8s = jnp.concatenate(bots, axis=0)[:, None, :]

        pre = c0_ref[:][:, :, None]
        for h in range(HEADS):
            a_h = ah_ref[h:h + 1, :]
            c_h = ch_ref[h:h + 1, :]
            g = a_h * st + c_h
            vsel = jnp.where(g[None] > 0.0, t8s, b8s)
            z = g[None] * vsel
            e = jnp.exp(z)
            den = jnp.sum(e, axis=0)
            num = jnp.sum(e * vsel, axis=0)
            m_h = num / den
            pre = pre + m_h[None] * at_ref[:, h:h + 1][:, :, None]

        gact = _gelu(pre)
        z_t = jnp.sum(gact * wm2_ref[:][:, :, None], axis=0) + bm2_e
        prev = res + ALPHA * z_t.T.reshape(B, PATCH, C)
        blocks.append(prev)

    out_ref[:] = jnp.concatenate(blocks, axis=1)


@jax.jit
def kernel(x, ng, nb, n1g, n1b, n2g, n2b, Wa, ba, Wce, bce, Wq, bq,
           Wk, bk, Wv, bv, Wm1, bm1, Wm2, bm2):
    f32 = jnp.float32
    xt = jnp.transpose(x, (0, 2, 1))
    ng2 = ng.reshape(C, T).T
    nb2 = nb.reshape(C, T).T
    n1g2 = n1g.reshape(C, PATCH).T
    n1b2 = n1b.reshape(C, PATCH).T
    n2g2 = n2g.reshape(C, PATCH).T
    n2b2 = n2b.reshape(C, PATCH).T

    w = Wce[:, 0]
    uq = Wq @ w
    cq = Wq @ bce + bq
    uk = Wk @ w
    ck = Wk @ bce + bk
    uv = Wv @ w
    cv = Wv @ bce + bv
    scale = 1.0 / math.sqrt(HD)
    uqh = uq.reshape(HEADS, HD)
    ukh = uk.reshape(HEADS, HD)
    cqh = cq.reshape(HEADS, HD)
    uvh = uv.reshape(HEADS, HD)
    a_h = jnp.sum(uqh * ukh, axis=1) * scale
    c_h = jnp.sum(cqh * ukh, axis=1) * scale
    at = jnp.sum(Wm1.reshape(FF, HEADS, HD) * uvh[None], axis=2)
    c0 = Wm1 @ cv + bm1
    wm2 = Wm2[0]

    out = pl.pallas_call(
        _body,
        out_shape=jax.ShapeDtypeStruct((B, T, C), f32),
    )(xt.astype(f32), ng2, nb2, n1g2, n1b2, n2g2, n2b2,
      Wa, ba.reshape(PATCH, 1), a_h.reshape(HEADS, 1), c_h.reshape(HEADS, 1),
      at, c0.reshape(FF, 1), wm2.reshape(FF, 1), bm2.reshape(1, 1))
    return jnp.transpose(out, (0, 2, 1))
